# Initial kernel scaffold; baseline (speedup 1.0000x reference)
import jax
import jax.numpy as jnp
from jax.experimental import pallas as pl


def kernel(feats, max_len):
    raise NotImplementedError("write your pallas kernel here")


# speedup vs baseline: 2.6785x; 2.6785x over previous
import jax
import jax.numpy as jnp
from jax.experimental import pallas as pl

_BATCH = 1024
_L = 200
_D = 128
_RATIO = 0.25
_BB = 16


def _body(mask_ref, x_ref, o_ref):
    x = x_ref[...]
    m = mask_ref[...]
    xm1 = jnp.concatenate([x[:, :1, :], x[:, :-1, :]], axis=1)
    avg = 0.5 * (x + xm1)
    avg_sh = jnp.concatenate([avg[:, :1, :], avg[:, :-1, :]], axis=1)
    t = jax.lax.broadcasted_iota(jnp.int32, x.shape, 1)
    pooled = jnp.where((t % 2) == 0, avg, avg_sh)
    o_ref[...] = m * pooled + (1.0 - m) * x


def kernel(feats, max_len):
    batch = feats.shape[0]
    num_to_pool = int(batch * _RATIO)
    perm = jax.random.permutation(jax.random.key(1), batch)
    ind = jnp.sort(perm[:num_to_pool])
    mask = jnp.zeros((batch,), jnp.float32).at[ind].set(1.0)
    mask3 = jnp.broadcast_to(mask[:, None, None], (batch, 1, _D))

    return pl.pallas_call(
        _body,
        grid=(batch // _BB,),
        in_specs=[
            pl.BlockSpec((_BB, 1, _D), lambda i: (i, 0, 0)),
            pl.BlockSpec((_BB, _L, _D), lambda i: (i, 0, 0)),
        ],
        out_specs=pl.BlockSpec((_BB, _L, _D), lambda i: (i, 0, 0)),
        out_shape=jax.ShapeDtypeStruct(feats.shape, feats.dtype),
    )(mask3, feats)


# speedup vs baseline: 3.1960x; 1.1932x over previous
import jax
import jax.numpy as jnp
from jax.experimental import pallas as pl
from jax.experimental.pallas import tpu as pltpu

_BATCH = 1024
_L = 200
_D = 128
_RATIO = 0.25
_BB = 32


def _body(mask_ref, x_ref, o_ref):
    x = x_ref[...]
    m = mask_ref[...]
    xm1 = jnp.concatenate([x[:, :1, :], x[:, :-1, :]], axis=1)
    avg = 0.5 * (x + xm1)
    avg_sh = jnp.concatenate([avg[:, :1, :], avg[:, :-1, :]], axis=1)
    t = jax.lax.broadcasted_iota(jnp.int32, x.shape, 1)
    pooled = jnp.where((t % 2) == 0, avg, avg_sh)
    o_ref[...] = m * pooled + (1.0 - m) * x


def kernel(feats, max_len):
    batch = feats.shape[0]
    num_to_pool = int(batch * _RATIO)
    perm = jax.random.permutation(jax.random.key(1), batch)
    ind = jnp.sort(perm[:num_to_pool])
    mask = jnp.zeros((batch,), jnp.float32).at[ind].set(1.0)
    mask3 = jnp.broadcast_to(mask[:, None, None], (batch, 1, _D))

    return pl.pallas_call(
        _body,
        grid=(batch // _BB,),
        in_specs=[
            pl.BlockSpec((_BB, 1, _D), lambda i: (i, 0, 0)),
            pl.BlockSpec((_BB, _L, _D), lambda i: (i, 0, 0)),
        ],
        out_specs=pl.BlockSpec((_BB, _L, _D), lambda i: (i, 0, 0)),
        out_shape=jax.ShapeDtypeStruct(feats.shape, feats.dtype),
        compiler_params=pltpu.CompilerParams(
            dimension_semantics=("parallel",),
        ),
    )(mask3, feats)
